# baseline (speedup 1.0000x reference)
import jax
import jax.numpy as jnp
from jax.experimental import pallas as pl
from jax.experimental.pallas import tpu as pltpu

F32 = jnp.float32
BETA = 0.25


def _k1_body(x_ref, w_ref, b_ref, o_ref):
    od = pl.program_id(0)
    acc = jnp.zeros((2048, 64), F32)
    t = 0
    for td in range(2):
        for th in range(2):
            for tw in range(2):
                xs = x_ref[:, pl.ds(od + td, 1), th:th + 32,
                           tw:tw + 32, :].reshape(2048, 8)
                acc = acc + jnp.dot(xs, w_ref[t], preferred_element_type=F32)
                t += 1
    o_ref[:, 0] = jnp.maximum(acc + b_ref[0], 0.0).reshape(2, 1024, 64)


def _conv1(xb, w1r, b1):
    out = pl.pallas_call(
        _k1_body,
        grid=(32,),
        in_specs=[
            pl.BlockSpec((2, 33, 33, 33, 8), lambda od: (0, 0, 0, 0, 0)),
            pl.BlockSpec((8, 8, 64), lambda od: (0, 0, 0)),
            pl.BlockSpec((1, 64), lambda od: (0, 0)),
        ],
        out_specs=pl.BlockSpec((2, 1, 1024, 64), lambda od: (0, od, 0, 0)),
        out_shape=jax.ShapeDtypeStruct((2, 32, 1024, 64), F32),
        compiler_params=pltpu.CompilerParams(
            dimension_semantics=("parallel",)),
    )(xb, w1r, b1)
    return out


def _k2_body(x_ref, w_ref, b_ref, o_ref):
    od = pl.program_id(0)
    acc = jnp.zeros((512, 128), F32)
    t = 0
    for td in range(2):
        for th in range(2):
            for tw in range(2):
                xs = x_ref[:, pl.ds(od + td, 1), th:th + 16,
                           tw:tw + 16, :].reshape(512, 512)
                acc = acc + jnp.dot(xs, w_ref[t], preferred_element_type=F32)
                t += 1
    o_ref[:, 0] = jnp.maximum(acc + b_ref[0], 0.0).reshape(2, 256, 128)


def _conv2(xb, w2r, b2):
    out = pl.pallas_call(
        _k2_body,
        grid=(16,),
        in_specs=[
            pl.BlockSpec((2, 17, 17, 17, 512), lambda od: (0, 0, 0, 0, 0)),
            pl.BlockSpec((8, 512, 128), lambda od: (0, 0, 0)),
            pl.BlockSpec((1, 128), lambda od: (0, 0)),
        ],
        out_specs=pl.BlockSpec((2, 1, 256, 128), lambda od: (0, od, 0, 0)),
        out_shape=jax.ShapeDtypeStruct((2, 16, 256, 128), F32),
        compiler_params=pltpu.CompilerParams(
            dimension_semantics=("parallel",)),
    )(xb, w2r, b2)
    return out


def _k3_body(x_ref, wqc_ref, bqc_ref, cbt_ref, cb_ref, wpqc_ref, bpqc_ref,
             o_ref, idx_ref, loss_ref):
    r = pl.program_id(0)
    z = jnp.dot(x_ref[...], wqc_ref[...], preferred_element_type=F32) + bqc_ref[0]
    dot = jnp.dot(z, cbt_ref[...], preferred_element_type=F32)
    znorm = jnp.sum(z * z, axis=1, keepdims=True)
    cnorm = jnp.sum(cbt_ref[...] * cbt_ref[...], axis=0, keepdims=True)
    dist = (znorm + cnorm) - 2.0 * dot
    m = jnp.min(dist, axis=1, keepdims=True)
    iota = jax.lax.broadcasted_iota(jnp.int32, (1024, 512), 1)
    idx = jnp.min(jnp.where(dist <= m, iota, 512), axis=1, keepdims=True)
    idx_ref[...] = idx
    onehot = (iota == idx).astype(F32)
    zq = jnp.dot(onehot, cb_ref[...], preferred_element_type=F32)
    d2 = z - zq
    ssq = jnp.sum(d2 * d2, axis=None, keepdims=True)

    @pl.when(r == 0)
    def _():
        loss_ref[...] = jnp.zeros_like(loss_ref)

    loss_ref[...] += ssq

    @pl.when(r == 7)
    def _():
        loss_ref[...] = (1.0 + BETA) * (loss_ref[...] / (8192.0 * 128.0))

    zq_st = z + (zq - z)
    o_ref[...] = (jnp.dot(zq_st, wpqc_ref[...], preferred_element_type=F32)
                  + bpqc_ref[0])


def _vq(h2, wqc, bqc, cbt, cb, wpqc, bpqc):
    return pl.pallas_call(
        _k3_body,
        grid=(8,),
        in_specs=[
            pl.BlockSpec((1024, 128), lambda r: (r, 0)),
            pl.BlockSpec((128, 128), lambda r: (0, 0)),
            pl.BlockSpec((1, 128), lambda r: (0, 0)),
            pl.BlockSpec((128, 512), lambda r: (0, 0)),
            pl.BlockSpec((512, 128), lambda r: (0, 0)),
            pl.BlockSpec((128, 128), lambda r: (0, 0)),
            pl.BlockSpec((1, 128), lambda r: (0, 0)),
        ],
        out_specs=(
            pl.BlockSpec((1024, 128), lambda r: (r, 0)),
            pl.BlockSpec((1024, 1), lambda r: (r, 0)),
            pl.BlockSpec((1, 1), lambda r: (0, 0)),
        ),
        out_shape=(
            jax.ShapeDtypeStruct((8192, 128), F32),
            jax.ShapeDtypeStruct((8192, 1), jnp.int32),
            jax.ShapeDtypeStruct((1, 1), F32),
        ),
        compiler_params=pltpu.CompilerParams(
            dimension_semantics=("arbitrary",)),
    )(h2, wqc, bqc, cbt, cb, wpqc, bpqc)


def _k4_body(x_ref, w_ref, b_ref, o_ref):
    p = pl.program_id(0)
    pd, ph, pw = p // 4, (p // 2) % 2, p % 2
    acc = jnp.zeros((8192, 64), F32)
    t = 0
    for ad in range(2):
        for ah in range(2):
            for aw in range(2):
                xs = x_ref[:, pl.ds(pd + ad, 16), pl.ds(ph + ah, 16),
                           pl.ds(pw + aw, 16), :].reshape(8192, 128)
                acc = acc + jnp.dot(xs, w_ref[0, t], preferred_element_type=F32)
                t += 1
    o_ref[0] = jnp.maximum(acc + b_ref[0], 0.0)


def _deconv1(xp, w4r, b4):
    return pl.pallas_call(
        _k4_body,
        grid=(8,),
        in_specs=[
            pl.BlockSpec((2, 18, 18, 18, 128), lambda p: (0, 0, 0, 0, 0)),
            pl.BlockSpec((1, 8, 128, 64), lambda p: (p, 0, 0, 0)),
            pl.BlockSpec((1, 64), lambda p: (0, 0)),
        ],
        out_specs=pl.BlockSpec((1, 8192, 64), lambda p: (p, 0, 0)),
        out_shape=jax.ShapeDtypeStruct((8, 8192, 64), F32),
        compiler_params=pltpu.CompilerParams(
            dimension_semantics=("parallel",)),
    )(xp, w4r, b4)


def _rb_body(x_ref, o_ref):
    pd = pl.program_id(0)
    odp = pl.program_id(1)
    cols = []
    for pw in range(2):
        rows = [x_ref[ph * 2 + pw, :, 0] for ph in range(2)]
        v = jnp.stack(rows, axis=2).reshape(2, 32, 16, 64)
        cols.append(v)
    v = jnp.stack(cols, axis=3).reshape(2, 32, 32, 64)
    v = jnp.pad(v, ((0, 0), (1, 1), (1, 1), (0, 0)))
    v = jnp.where(odp < 16, v, 0.0)
    o_ref[:, 0] = v


def _rb(h3p8):
    return pl.pallas_call(
        _rb_body,
        grid=(2, 17),
        in_specs=[
            pl.BlockSpec((4, 2, 1, 16, 16, 64),
                         lambda pd, odp: (pd, 0, jnp.minimum(odp, 15), 0, 0, 0)),
        ],
        out_specs=pl.BlockSpec(
            (2, 1, 34, 34, 64),
            lambda pd, odp: (0, jnp.where(odp < 16, 2 * odp + 1 + pd,
                                          33 * (1 - pd)), 0, 0, 0)),
        out_shape=jax.ShapeDtypeStruct((2, 34, 34, 34, 64), F32),
        compiler_params=pltpu.CompilerParams(
            dimension_semantics=("parallel", "parallel")),
    )(h3p8)


def _k5_body(x_ref, w_ref, b_ref, o_ref):
    od = pl.program_id(0)
    acc = jnp.zeros((2048, 8), F32)
    t = 0
    for sd in range(3):
        for sh in range(3):
            for sw in range(3):
                xs = x_ref[:, pl.ds(od + sd, 1), sh:sh + 32,
                           sw:sw + 32, :].reshape(2048, 64)
                acc = acc + jnp.dot(xs, w_ref[t], preferred_element_type=F32)
                t += 1
    o_ref[0] = acc + b_ref[...]


def _deconv2(xp, wc, b5):
    return pl.pallas_call(
        _k5_body,
        grid=(32,),
        in_specs=[
            pl.BlockSpec((2, 34, 34, 34, 64), lambda od: (0, 0, 0, 0, 0)),
            pl.BlockSpec((27, 64, 8), lambda od: (0, 0, 0)),
            pl.BlockSpec((1, 1), lambda od: (0, 0)),
        ],
        out_specs=pl.BlockSpec((1, 2048, 8), lambda od: (od, 0, 0)),
        out_shape=jax.ShapeDtypeStruct((32, 2048, 8), F32),
        compiler_params=pltpu.CompilerParams(
            dimension_semantics=("parallel",)),
    )(xp, wc, b5)


def _s2d_in(x, blocks):
    B, S, _, _, C = x.shape
    xp = jnp.pad(x, ((0, 0), (1, 1), (1, 1), (1, 1), (0, 0)))
    xp = xp.reshape(B, blocks, 2, blocks, 2, blocks, 2, C)
    xp = xp.transpose(0, 1, 3, 5, 2, 4, 6, 7)
    return xp.reshape(B, blocks, blocks, blocks, 8 * C)


def _fwd_weight(w):
    O, I = w.shape[0], w.shape[1]
    w = w.reshape(O, I, 2, 2, 2, 2, 2, 2)
    w = w.transpose(2, 4, 6, 3, 5, 7, 1, 0)
    return w.reshape(8, 8 * I, O)


def _bwd_weight(w):
    O, I = w.shape[0], w.shape[1]
    w = w.reshape(O, I, 2, 2, 2, 2, 2, 2)
    w = w.transpose(3, 5, 7, 2, 4, 6, 1, 0)
    return w.reshape(8, 8, I, O)


def _k5_weight(w):
    cols = []
    for sd in range(3):
        for sh in range(3):
            for sw in range(3):
                pcols = []
                for pd in range(2):
                    for ph in range(2):
                        for pw in range(2):
                            ok = (0 <= sd - pd <= 1 and 0 <= sh - ph <= 1
                                  and 0 <= sw - pw <= 1)
                            if ok:
                                pcols.append(w[0, :, 2 * sd - pd,
                                               2 * sh - ph, 2 * sw - pw])
                            else:
                                pcols.append(jnp.zeros((64,), F32))
                cols.append(jnp.stack(pcols, axis=1))
    return jnp.stack(cols, axis=0)


def _interleave(y, B, S, C):
    y = y.reshape(2, 2, 2, B, S, S, S, C)
    y = y.transpose(3, 4, 0, 5, 1, 6, 2, 7)
    return y.reshape(B, 2 * S, 2 * S, 2 * S, C)


def kernel(imgs, enc_w1, enc_b1, enc_w2, enc_b2, qc_w, qc_b, codebook,
           pqc_w, pqc_b, dec_w1, dec_b1, dec_w2, dec_b2):
    B = imgs.shape[0]
    x = imgs.transpose(0, 2, 3, 4, 1)
    xb = _s2d_in(x, 33)
    h1 = _conv1(xb, _fwd_weight(enc_w1), enc_b1.reshape(1, 64))
    h1 = h1.reshape(B, 32, 32, 32, 64)
    h1b = _s2d_in(h1, 17)
    h2 = _conv2(h1b, _fwd_weight(enc_w2), enc_b2.reshape(1, 128))
    h2 = h2.reshape(B * 4096, 128)
    wqc = qc_w[:, :, 0, 0, 0].T
    wpqc = pqc_w[:, :, 0, 0, 0].T
    zqp, idx, qloss = _vq(h2, wqc, qc_b.reshape(1, 128), codebook.T,
                          codebook, wpqc, pqc_b.reshape(1, 128))
    codebook_indices = idx.reshape(B, 16, 16, 16)
    q_loss = qloss[0, 0]
    q5 = zqp.reshape(B, 16, 16, 16, 128)
    qp = jnp.pad(q5, ((0, 0), (1, 1), (1, 1), (1, 1), (0, 0)))
    h3 = _deconv1(qp, _bwd_weight(dec_w1), dec_b1.reshape(1, 64))
    h3p = _rb(h3.reshape(8, B, 16, 16, 16, 64))
    y = _deconv2(h3p, _k5_weight(dec_w2), dec_b2.reshape(1, 1))
    y = y.reshape(32, B, 32, 32, 2, 2, 2)
    y = y.transpose(1, 0, 4, 2, 5, 3, 6).reshape(B, 64, 64, 64)
    decoded_images = y[:, None]
    return decoded_images, codebook_indices, q_loss


# speedup vs baseline: 1.2507x; 1.2507x over previous
import jax
import jax.numpy as jnp
from jax.experimental import pallas as pl
from jax.experimental.pallas import tpu as pltpu

F32 = jnp.float32
BETA = 0.25


def _k1_body(x_ref, w_ref, b_ref, o_ref):
    pd = pl.program_id(0)
    dblk = pl.program_id(1)
    od = 2 * dblk + pd - 1
    valid = jnp.logical_and(od >= 0, od <= 31)
    odc = jnp.clip(od, 0, 31)
    acc = jnp.zeros((2048, 64), F32)
    t = 0
    for td in range(2):
        for th in range(2):
            for tw in range(2):
                xs = x_ref[:, pl.ds(odc + td, 1), th:th + 32,
                           tw:tw + 32, :].reshape(2048, 8)
                acc = acc + jnp.dot(xs, w_ref[t], preferred_element_type=F32)
                t += 1
    r = jnp.maximum(acc + b_ref[0], 0.0).reshape(2, 32, 32, 64)
    r = jnp.where(valid, r, 0.0)
    o_ref[0, :, 0] = jnp.zeros((2, 17, 17, 256), F32)
    rs = r.reshape(2, 16, 2, 16, 2, 64)
    for hp in range(2):
        for wp in range(2):
            v = rs[:, :, 1 - hp, :, 1 - wp, :]
            hs, ws, c0 = 1 - hp, 1 - wp, (hp * 2 + wp) * 64
            o_ref[0, :, 0, hs:hs + 16, ws:ws + 16, c0:c0 + 64] = v


def _conv1(xb, w1r, b1):
    return pl.pallas_call(
        _k1_body,
        grid=(2, 17),
        in_specs=[
            pl.BlockSpec((2, 33, 33, 33, 8), lambda pd, j: (0, 0, 0, 0, 0)),
            pl.BlockSpec((8, 8, 64), lambda pd, j: (0, 0, 0)),
            pl.BlockSpec((1, 64), lambda pd, j: (0, 0)),
        ],
        out_specs=pl.BlockSpec((1, 2, 1, 17, 17, 256),
                               lambda pd, j: (pd, 0, j, 0, 0, 0)),
        out_shape=jax.ShapeDtypeStruct((2, 2, 17, 17, 17, 256), F32),
        compiler_params=pltpu.CompilerParams(
            dimension_semantics=("parallel", "parallel")),
    )(xb, w1r, b1)


def _k2_body(x_ref, w_ref, b_ref, o_ref):
    od = pl.program_id(0)
    acc = jnp.zeros((512, 128), F32)
    t = 0
    for td in range(2):
        for th in range(2):
            for tw in range(2):
                for pd in range(2):
                    xs = x_ref[pd, :, pl.ds(od + td, 1), th:th + 16,
                               tw:tw + 16, :].reshape(512, 256)
                    acc = acc + jnp.dot(xs, w_ref[t, pd],
                                        preferred_element_type=F32)
                t += 1
    o_ref[:, 0] = jnp.maximum(acc + b_ref[0], 0.0).reshape(2, 256, 128)


def _conv2(xb, w2r, b2):
    return pl.pallas_call(
        _k2_body,
        grid=(16,),
        in_specs=[
            pl.BlockSpec((2, 2, 17, 17, 17, 256),
                         lambda od: (0, 0, 0, 0, 0, 0)),
            pl.BlockSpec((8, 2, 256, 128), lambda od: (0, 0, 0, 0)),
            pl.BlockSpec((1, 128), lambda od: (0, 0)),
        ],
        out_specs=pl.BlockSpec((2, 1, 256, 128), lambda od: (0, od, 0, 0)),
        out_shape=jax.ShapeDtypeStruct((2, 16, 256, 128), F32),
        compiler_params=pltpu.CompilerParams(
            dimension_semantics=("parallel",)),
    )(xb, w2r, b2)


def _k3_body(x_ref, wqc_ref, bqc_ref, cbt_ref, cb_ref, wpqc_ref, bpqc_ref,
             o_ref, idx_ref, loss_ref):
    r = pl.program_id(0)
    z = jnp.dot(x_ref[...], wqc_ref[...], preferred_element_type=F32) + bqc_ref[0]
    dot = jnp.dot(z, cbt_ref[...], preferred_element_type=F32)
    znorm = jnp.sum(z * z, axis=1, keepdims=True)
    cnorm = jnp.sum(cbt_ref[...] * cbt_ref[...], axis=0, keepdims=True)
    dist = (znorm + cnorm) - 2.0 * dot
    m = jnp.min(dist, axis=1, keepdims=True)
    iota = jax.lax.broadcasted_iota(jnp.int32, (1024, 512), 1)
    idx = jnp.min(jnp.where(dist <= m, iota, 512), axis=1, keepdims=True)
    idx_ref[...] = idx
    onehot = (iota == idx).astype(F32)
    zq = jnp.dot(onehot, cb_ref[...], preferred_element_type=F32)
    d2 = z - zq
    ssq = jnp.sum(d2 * d2, axis=None, keepdims=True)

    @pl.when(r == 0)
    def _():
        loss_ref[...] = jnp.zeros_like(loss_ref)

    loss_ref[...] += ssq

    @pl.when(r == 7)
    def _():
        loss_ref[...] = (1.0 + BETA) * (loss_ref[...] / (8192.0 * 128.0))

    zq_st = z + (zq - z)
    o_ref[...] = (jnp.dot(zq_st, wpqc_ref[...], preferred_element_type=F32)
                  + bpqc_ref[0])


def _vq(h2, wqc, bqc, cbt, cb, wpqc, bpqc):
    return pl.pallas_call(
        _k3_body,
        grid=(8,),
        in_specs=[
            pl.BlockSpec((1024, 128), lambda r: (r, 0)),
            pl.BlockSpec((128, 128), lambda r: (0, 0)),
            pl.BlockSpec((1, 128), lambda r: (0, 0)),
            pl.BlockSpec((128, 512), lambda r: (0, 0)),
            pl.BlockSpec((512, 128), lambda r: (0, 0)),
            pl.BlockSpec((128, 128), lambda r: (0, 0)),
            pl.BlockSpec((1, 128), lambda r: (0, 0)),
        ],
        out_specs=(
            pl.BlockSpec((1024, 128), lambda r: (r, 0)),
            pl.BlockSpec((1024, 1), lambda r: (r, 0)),
            pl.BlockSpec((1, 1), lambda r: (0, 0)),
        ),
        out_shape=(
            jax.ShapeDtypeStruct((8192, 128), F32),
            jax.ShapeDtypeStruct((8192, 1), jnp.int32),
            jax.ShapeDtypeStruct((1, 1), F32),
        ),
        compiler_params=pltpu.CompilerParams(
            dimension_semantics=("arbitrary",)),
    )(h2, wqc, bqc, cbt, cb, wpqc, bpqc)


def _k4_body(x_ref, w_ref, b_ref, o_ref):
    X = pl.program_id(0)
    Xc = jnp.clip(X, 1, 32)
    od = (Xc - 1) // 2
    pd = (Xc - 1) % 2
    valid = jnp.logical_and(X >= 1, X <= 32)
    rs = []
    for ph in range(2):
        for pw in range(2):
            acc = jnp.zeros((512, 64), F32)
            t = 0
            for ad in range(2):
                for ah in range(2):
                    for aw in range(2):
                        xs = x_ref[:, pl.ds(od + pd + ad, 1),
                                   pl.ds(ph + ah, 16), pl.ds(pw + aw, 16),
                                   :].reshape(512, 128)
                        wt = w_ref[pd * 4 + ph * 2 + pw, t]
                        acc = acc + jnp.dot(xs, wt, preferred_element_type=F32)
                        t += 1
            rs.append(jnp.maximum(acc + b_ref[0], 0.0).reshape(2, 16, 16, 64))
    u0 = jnp.stack([rs[0], rs[1]], axis=3).reshape(2, 16, 32, 64)
    u1 = jnp.stack([rs[2], rs[3]], axis=3).reshape(2, 16, 32, 64)
    v = jnp.stack([u0, u1], axis=2).reshape(2, 32, 32, 64)
    v = jnp.pad(v, ((0, 0), (1, 1), (1, 1), (0, 0)))
    v = jnp.where(valid, v, 0.0)
    o_ref[:, 0] = v


def _deconv1(xp, w4r, b4):
    return pl.pallas_call(
        _k4_body,
        grid=(34,),
        in_specs=[
            pl.BlockSpec((2, 18, 18, 18, 128), lambda X: (0, 0, 0, 0, 0)),
            pl.BlockSpec((8, 8, 128, 64), lambda X: (0, 0, 0, 0)),
            pl.BlockSpec((1, 64), lambda X: (0, 0)),
        ],
        out_specs=pl.BlockSpec((2, 1, 34, 34, 64), lambda X: (0, X, 0, 0, 0)),
        out_shape=jax.ShapeDtypeStruct((2, 34, 34, 34, 64), F32),
        compiler_params=pltpu.CompilerParams(
            dimension_semantics=("parallel",)),
    )(xp, w4r, b4)


def _k5_body(x_ref, w_ref, b_ref, o_ref):
    od = pl.program_id(0)
    acc = jnp.zeros((2048, 8), F32)
    t = 0
    for sd in range(3):
        for sh in range(3):
            for sw in range(3):
                xs = x_ref[:, pl.ds(od + sd, 1), sh:sh + 32,
                           sw:sw + 32, :].reshape(2048, 64)
                acc = acc + jnp.dot(xs, w_ref[t], preferred_element_type=F32)
                t += 1
    o_ref[0] = acc + b_ref[...]


def _deconv2(xp, wc, b5):
    return pl.pallas_call(
        _k5_body,
        grid=(32,),
        in_specs=[
            pl.BlockSpec((2, 34, 34, 34, 64), lambda od: (0, 0, 0, 0, 0)),
            pl.BlockSpec((27, 64, 8), lambda od: (0, 0, 0)),
            pl.BlockSpec((1, 1), lambda od: (0, 0)),
        ],
        out_specs=pl.BlockSpec((1, 2048, 8), lambda od: (od, 0, 0)),
        out_shape=jax.ShapeDtypeStruct((32, 2048, 8), F32),
        compiler_params=pltpu.CompilerParams(
            dimension_semantics=("parallel",)),
    )(xp, wc, b5)


def _s2d_in(x, blocks):
    B, S, _, _, C = x.shape
    xp = jnp.pad(x, ((0, 0), (1, 1), (1, 1), (1, 1), (0, 0)))
    xp = xp.reshape(B, blocks, 2, blocks, 2, blocks, 2, C)
    xp = xp.transpose(0, 1, 3, 5, 2, 4, 6, 7)
    return xp.reshape(B, blocks, blocks, blocks, 8 * C)


def _fwd_weight(w):
    O, I = w.shape[0], w.shape[1]
    w = w.reshape(O, I, 2, 2, 2, 2, 2, 2)
    w = w.transpose(2, 4, 6, 3, 5, 7, 1, 0)
    return w.reshape(8, 8 * I, O)


def _bwd_weight(w):
    O, I = w.shape[0], w.shape[1]
    w = w.reshape(O, I, 2, 2, 2, 2, 2, 2)
    w = w.transpose(3, 5, 7, 2, 4, 6, 1, 0)
    return w.reshape(8, 8, I, O)


def _k5_weight(w):
    cols = []
    for sd in range(3):
        for sh in range(3):
            for sw in range(3):
                pcols = []
                for pd in range(2):
                    for ph in range(2):
                        for pw in range(2):
                            ok = (0 <= sd - pd <= 1 and 0 <= sh - ph <= 1
                                  and 0 <= sw - pw <= 1)
                            if ok:
                                pcols.append(w[0, :, 2 * sd - pd,
                                               2 * sh - ph, 2 * sw - pw])
                            else:
                                pcols.append(jnp.zeros((64,), F32))
                cols.append(jnp.stack(pcols, axis=1))
    return jnp.stack(cols, axis=0)


def kernel(imgs, enc_w1, enc_b1, enc_w2, enc_b2, qc_w, qc_b, codebook,
           pqc_w, pqc_b, dec_w1, dec_b1, dec_w2, dec_b2):
    B = imgs.shape[0]
    x = imgs.transpose(0, 2, 3, 4, 1)
    xb = _s2d_in(x, 33)
    h1b = _conv1(xb, _fwd_weight(enc_w1), enc_b1.reshape(1, 64))
    w2r = _fwd_weight(enc_w2).reshape(8, 2, 256, 128)
    h2 = _conv2(h1b, w2r, enc_b2.reshape(1, 128))
    h2 = h2.reshape(B * 4096, 128)
    wqc = qc_w[:, :, 0, 0, 0].T
    wpqc = pqc_w[:, :, 0, 0, 0].T
    zqp, idx, qloss = _vq(h2, wqc, qc_b.reshape(1, 128), codebook.T,
                          codebook, wpqc, pqc_b.reshape(1, 128))
    codebook_indices = idx.reshape(B, 16, 16, 16)
    q_loss = qloss[0, 0]
    q5 = zqp.reshape(B, 16, 16, 16, 128)
    qp = jnp.pad(q5, ((0, 0), (1, 1), (1, 1), (1, 1), (0, 0)))
    h3p = _deconv1(qp, _bwd_weight(dec_w1), dec_b1.reshape(1, 64))
    y = _deconv2(h3p, _k5_weight(dec_w2), dec_b2.reshape(1, 1))
    y = y.reshape(32, B, 32, 32, 2, 2, 2)
    y = y.transpose(1, 0, 4, 2, 5, 3, 6).reshape(B, 64, 64, 64)
    decoded_images = y[:, None]
    return decoded_images, codebook_indices, q_loss
